# baseline (speedup 1.0000x reference)
import jax
import jax.numpy as jnp
from jax import lax
from jax.experimental import pallas as pl

_N = 16384
_C = 1000
_BR = 512
_NB = _N // _BR


def _nll_block(prob_ref, tgt_ref, w_ref, out_ref):
    t = tgt_ref[0, :, :]
    col = lax.broadcasted_iota(jnp.int32, (_BR, _C), 1)
    masked = jnp.where(col == t, prob_ref[...], 0.0)
    s = jnp.sum(masked, axis=0, keepdims=True)
    out_ref[...] = jnp.sum(s * w_ref[...]).reshape(1, 1, 1)


_nll_partials = pl.pallas_call(
    _nll_block,
    grid=(_NB,),
    in_specs=[
        pl.BlockSpec((_BR, _C), lambda i: (i, 0)),
        pl.BlockSpec((1, _BR, 1), lambda i: (i, 0, 0)),
        pl.BlockSpec((1, _C), lambda i: (0, 0)),
    ],
    out_specs=pl.BlockSpec((1, 1, 1), lambda i: (i, 0, 0)),
    out_shape=jax.ShapeDtypeStruct((_NB, 1, 1), jnp.float32),
)


def kernel(prob, target, weight):
    tgt_3d = target.reshape(_NB, _BR, 1)
    partials = _nll_partials(prob, tgt_3d, weight.reshape(1, _C))
    return -jnp.sum(partials)
